# baseline (speedup 1.0000x reference)
import jax
import jax.numpy as jnp
from jax import lax
from jax.experimental import pallas as pl
from jax.experimental.pallas import tpu as pltpu
from jax.experimental.pallas import tpu_sc as plsc

N_NODES = 10000
N_EDGES = 320000
FDIM = 128

NC = 2
NS = 16
NW = NC * NS
EPW = N_EDGES // NW
CHUNK = 128
EPW_PAD = 10112
NPAD = EPW_PAD - EPW
NCHUNK = EPW_PAD // CHUNK
N_ACC = 10240
ROWS_PER_TILE = N_ACC // NS
ZROWS = 8
LANES = 16

_MESH = plsc.VectorSubcoreMesh(
    core_axis_name="c", subcore_axis_name="s", num_cores=NC, num_subcores=NS
)


def _sc_aggregate_build():
    scratch = [
        pltpu.VMEM((NCHUNK, CHUNK), jnp.int32),
        pltpu.VMEM((NCHUNK, CHUNK), jnp.int32),
        pltpu.VMEM((CHUNK, FDIM), jnp.float32),
        pltpu.VMEM((ZROWS, FDIM), jnp.float32),
        pltpu.VMEM_SHARED((N_ACC, FDIM), jnp.float32),
        pltpu.SemaphoreType.DMA,
    ]

    def body(h_hbm, src_hbm, dst_hbm, out_agg,
             srcv, dstv, rows0, zbuf, aggsh, sem0):
        cid = lax.axis_index("c")
        sid = lax.axis_index("s")
        wid = sid * NC + cid

        zf = jnp.zeros((LANES,), jnp.float32)
        per_row = FDIM // LANES

        def zero_zbuf(t, carry):
            zbuf[t // per_row, pl.ds((t % per_row) * LANES, LANES)] = zf
            return carry

        lax.fori_loop(0, ZROWS * per_row, zero_zbuf, 0)

        base = sid * ROWS_PER_TILE

        def zrow(r, carry):
            off = pl.multiple_of(base + r * ZROWS, ZROWS)
            pltpu.sync_copy(zbuf, aggsh.at[pl.ds(off, ZROWS)])
            return carry

        lax.fori_loop(0, ROWS_PER_TILE // ZROWS, zrow, 0)

        pltpu.sync_copy(src_hbm.at[wid], srcv)
        pltpu.sync_copy(dst_hbm.at[wid], dstv)

        plsc.subcore_barrier()

        def chunk_body(k, carry):
            pltpu.async_copy(h_hbm.at[srcv.at[k]], rows0, sem0).wait()
            pltpu.sync_copy(rows0, aggsh.at[dstv.at[k]], add=True)
            return carry

        lax.fori_loop(0, NCHUNK, chunk_body, 0)

        plsc.subcore_barrier()

        pltpu.sync_copy(
            aggsh.at[pl.ds(base, ROWS_PER_TILE)],
            out_agg.at[cid, pl.ds(base, ROWS_PER_TILE)],
        )

    return pl.kernel(
        body,
        out_type=jax.ShapeDtypeStruct((NC, N_ACC, FDIM), jnp.float32),
        mesh=_MESH,
        scratch_types=scratch,
    )


_sc_aggregate = _sc_aggregate_build()


def _sc_count_build():
    scratch = [
        pltpu.VMEM((EPW // 80, 80), jnp.int32),
        pltpu.VMEM((N_ACC,), jnp.float32),
    ]

    def body(dst_hbm, out_cnt, dstv, cntv):
        cid = lax.axis_index("c")
        sid = lax.axis_index("s")
        wid = sid * NC + cid

        zf = jnp.zeros((LANES,), jnp.float32)

        def zero_cntv(t, carry):
            cntv[pl.ds(t * LANES, LANES)] = zf
            return carry

        lax.fori_loop(0, N_ACC // LANES, zero_cntv, 0)
        pltpu.sync_copy(dst_hbm.at[wid], dstv)

        def cnt_body(t, carry):
            d = dstv[t // (80 // LANES),
                     pl.ds((t % (80 // LANES)) * LANES, LANES)]
            rcount, last = plsc.scan_count(d)
            plsc.addupdate_scatter(
                cntv, [d], rcount.astype(jnp.float32), mask=last
            )
            return carry

        lax.fori_loop(0, EPW // LANES, cnt_body, 0)

        def cnt_wb(r, carry):
            off = pl.multiple_of(r * 1024, 8)
            pltpu.sync_copy(cntv.at[pl.ds(off, 1024)],
                            out_cnt.at[wid, pl.ds(off, 1024)])
            return carry

        lax.fori_loop(0, N_ACC // 1024, cnt_wb, 0)

    return pl.kernel(
        body,
        out_type=jax.ShapeDtypeStruct((NW, N_ACC), jnp.float32),
        mesh=_MESH,
        scratch_types=scratch,
        compiler_params=pltpu.CompilerParams(needs_layout_passes=False),
    )


_sc_count = _sc_count_build()

BM = 1000


def _dense(x, w, b):
    m, k = x.shape
    o = w.shape[1]

    def body(x_ref, w_ref, b_ref, o_ref):
        o_ref[...] = (
            jnp.dot(x_ref[...], w_ref[...], preferred_element_type=jnp.float32)
            + b_ref[...]
        )

    return pl.pallas_call(
        body,
        grid=(m // BM,),
        in_specs=[
            pl.BlockSpec((BM, k), lambda i: (i, 0)),
            pl.BlockSpec((k, o), lambda i: (0, 0)),
            pl.BlockSpec((1, o), lambda i: (0, 0)),
        ],
        out_specs=pl.BlockSpec((BM, o), lambda i: (i, 0)),
        out_shape=jax.ShapeDtypeStruct((m, o), jnp.float32),
    )(x, w, b.reshape(1, o))


def _combine(aggp, cntp, h, wn, ws, bh, wo=None, bo=None):
    project = wo is not None

    def body(a_ref, c_ref, h_ref, wn_ref, ws_ref, b_ref, *rest):
        o_ref = rest[-1]
        ones_w = jnp.ones((NW, 1), jnp.float32)
        cnt = jnp.dot(c_ref[...], ones_w, preferred_element_type=jnp.float32)
        agg = (a_ref[0] + a_ref[1]) / jnp.maximum(cnt, 1.0)
        y = (
            jnp.dot(agg, wn_ref[...], preferred_element_type=jnp.float32)
            + jnp.dot(h_ref[...], ws_ref[...], preferred_element_type=jnp.float32)
            + b_ref[...]
        )
        t = jnp.maximum(y, 0.0)
        if project:
            wo_ref, bo_ref = rest[0], rest[1]
            t = (
                jnp.dot(t, wo_ref[...], preferred_element_type=jnp.float32)
                + bo_ref[...]
            )
        o_ref[...] = t

    in_specs = [
        pl.BlockSpec((NC, BM, FDIM), lambda i: (0, i, 0)),
        pl.BlockSpec((BM, NW), lambda i: (i, 0)),
        pl.BlockSpec((BM, FDIM), lambda i: (i, 0)),
        pl.BlockSpec((FDIM, FDIM), lambda i: (0, 0)),
        pl.BlockSpec((FDIM, FDIM), lambda i: (0, 0)),
        pl.BlockSpec((1, FDIM), lambda i: (0, 0)),
    ]
    args = [aggp, cntp, h, wn, ws, bh.reshape(1, FDIM)]
    odim = FDIM
    if project:
        odim = wo.shape[1]
        in_specs += [
            pl.BlockSpec((FDIM, odim), lambda i: (0, 0)),
            pl.BlockSpec((1, odim), lambda i: (0, 0)),
        ]
        args += [wo, bo.reshape(1, odim)]

    return pl.pallas_call(
        body,
        grid=(N_NODES // BM,),
        in_specs=in_specs,
        out_specs=pl.BlockSpec((BM, odim), lambda i: (i, 0)),
        out_shape=jax.ShapeDtypeStruct((N_NODES, odim), jnp.float32),
    )(*args)


def kernel(x, edge_index, W_in, b_in, W_neigh, W_self, b_hidden, W_out, b_out):
    ei = edge_index.astype(jnp.int32)
    dstw = ei[0].reshape(NW, EPW)
    srcw = ei[1].reshape(NW, EPW)
    pad_src = jnp.zeros((NW, NPAD), jnp.int32)
    pad_dst = N_NODES + (
        jnp.arange(NPAD, dtype=jnp.int32)[None, :]
        + 15 * jnp.arange(NW, dtype=jnp.int32)[:, None]
    ) % (N_ACC - N_NODES)
    dst3d = jnp.concatenate([dstw, pad_dst], axis=1).reshape(NW, NCHUNK, CHUNK)
    src3d = jnp.concatenate([srcw, pad_src], axis=1).reshape(NW, NCHUNK, CHUNK)
    dst_cnt = dstw.reshape(NW, EPW // 80, 80)

    h = _dense(x, W_in, b_in)
    cntp = _sc_count(dst_cnt)
    cnt_t = cntp.T
    aggp = _sc_aggregate(h, src3d, dst3d)
    h = _combine(aggp, cnt_t, h, W_neigh[0], W_self[0], b_hidden[0])
    aggp = _sc_aggregate(h, src3d, dst3d)
    return _combine(aggp, cnt_t, h, W_neigh[1], W_self[1], b_hidden[1],
                    W_out, b_out)


# speedup vs baseline: 1.3856x; 1.3856x over previous
import jax
import jax.numpy as jnp
from jax import lax
from jax.experimental import pallas as pl
from jax.experimental.pallas import tpu as pltpu
from jax.experimental.pallas import tpu_sc as plsc

N_NODES = 10000
N_EDGES = 320000
FDIM = 128

NC = 2
NS = 16
NW = NC * NS
EPW = N_EDGES // NW
CHUNK = 80
NCHUNK = EPW // CHUNK
N_ACC = 10240
ROWS_PER_TILE = N_ACC // NS
ZROWS = 8
LANES = 16

_MESH = plsc.VectorSubcoreMesh(
    core_axis_name="c", subcore_axis_name="s", num_cores=NC, num_subcores=NS
)


def _sc_aggregate_build():
    scratch = [
        pltpu.VMEM((NCHUNK, CHUNK), jnp.int32),
        pltpu.VMEM((NCHUNK, CHUNK), jnp.int32),
        pltpu.VMEM((CHUNK, FDIM), jnp.float32),
        pltpu.VMEM((ZROWS, FDIM), jnp.float32),
        pltpu.VMEM_SHARED((N_ACC, FDIM), jnp.float32),
        pltpu.SemaphoreType.DMA,
    ]

    def body(h_hbm, src_hbm, dst_hbm, out_agg,
             srcv, dstv, rows0, zbuf, aggsh, sem0):
        cid = lax.axis_index("c")
        sid = lax.axis_index("s")
        wid = sid * NC + cid

        zf = jnp.zeros((LANES,), jnp.float32)
        per_row = FDIM // LANES

        def zero_zbuf(t, carry):
            zbuf[t // per_row, pl.ds((t % per_row) * LANES, LANES)] = zf
            return carry

        lax.fori_loop(0, ZROWS * per_row, zero_zbuf, 0)

        base = sid * ROWS_PER_TILE

        def zrow(r, carry):
            off = pl.multiple_of(base + r * ZROWS, ZROWS)
            pltpu.sync_copy(zbuf, aggsh.at[pl.ds(off, ZROWS)])
            return carry

        lax.fori_loop(0, ROWS_PER_TILE // ZROWS, zrow, 0)

        pltpu.sync_copy(src_hbm.at[wid], srcv)
        pltpu.sync_copy(dst_hbm.at[wid], dstv)

        plsc.subcore_barrier()

        def chunk_body(k, carry):
            pltpu.async_copy(h_hbm.at[srcv.at[k]], rows0, sem0).wait()
            pltpu.sync_copy(rows0, aggsh.at[dstv.at[k]], add=True)
            return carry

        lax.fori_loop(0, NCHUNK, chunk_body, 0)

        plsc.subcore_barrier()

        pltpu.sync_copy(
            aggsh.at[pl.ds(base, ROWS_PER_TILE)],
            out_agg.at[cid, pl.ds(base, ROWS_PER_TILE)],
        )

    return pl.kernel(
        body,
        out_type=jax.ShapeDtypeStruct((NC, N_ACC, FDIM), jnp.float32),
        mesh=_MESH,
        scratch_types=scratch,
    )


_sc_aggregate = _sc_aggregate_build()


def _sc_count_build():
    scratch = [
        pltpu.VMEM((EPW // 80, 80), jnp.int32),
        pltpu.VMEM((N_ACC,), jnp.float32),
    ]

    def body(dst_hbm, out_cnt, dstv, cntv):
        cid = lax.axis_index("c")
        sid = lax.axis_index("s")
        wid = sid * NC + cid

        zf = jnp.zeros((LANES,), jnp.float32)

        def zero_cntv(t, carry):
            cntv[pl.ds(t * LANES, LANES)] = zf
            return carry

        lax.fori_loop(0, N_ACC // LANES, zero_cntv, 0)
        pltpu.sync_copy(dst_hbm.at[wid], dstv)

        def cnt_body(t, carry):
            d = dstv[t // (80 // LANES),
                     pl.ds((t % (80 // LANES)) * LANES, LANES)]
            rcount, last = plsc.scan_count(d)
            plsc.addupdate_scatter(
                cntv, [d], rcount.astype(jnp.float32), mask=last
            )
            return carry

        lax.fori_loop(0, EPW // LANES, cnt_body, 0)

        def cnt_wb(r, carry):
            off = pl.multiple_of(r * 1024, 8)
            pltpu.sync_copy(cntv.at[pl.ds(off, 1024)],
                            out_cnt.at[wid, pl.ds(off, 1024)])
            return carry

        lax.fori_loop(0, N_ACC // 1024, cnt_wb, 0)

    return pl.kernel(
        body,
        out_type=jax.ShapeDtypeStruct((NW, N_ACC), jnp.float32),
        mesh=_MESH,
        scratch_types=scratch,
        compiler_params=pltpu.CompilerParams(needs_layout_passes=False),
    )


_sc_count = _sc_count_build()

BM = 1000


def _dense(x, w, b):
    m, k = x.shape
    o = w.shape[1]

    def body(x_ref, w_ref, b_ref, o_ref):
        o_ref[...] = (
            jnp.dot(x_ref[...], w_ref[...], preferred_element_type=jnp.float32)
            + b_ref[...]
        )

    return pl.pallas_call(
        body,
        grid=(m // BM,),
        in_specs=[
            pl.BlockSpec((BM, k), lambda i: (i, 0)),
            pl.BlockSpec((k, o), lambda i: (0, 0)),
            pl.BlockSpec((1, o), lambda i: (0, 0)),
        ],
        out_specs=pl.BlockSpec((BM, o), lambda i: (i, 0)),
        out_shape=jax.ShapeDtypeStruct((m, o), jnp.float32),
    )(x, w, b.reshape(1, o))


def _combine(aggp, cntp, h, wn, ws, bh, wo=None, bo=None):
    project = wo is not None

    def body(a_ref, c_ref, h_ref, wn_ref, ws_ref, b_ref, *rest):
        o_ref = rest[-1]
        ones_w = jnp.ones((NW, 1), jnp.float32)
        cnt = jnp.dot(c_ref[...], ones_w, preferred_element_type=jnp.float32)
        agg = (a_ref[0] + a_ref[1]) / jnp.maximum(cnt, 1.0)
        y = (
            jnp.dot(agg, wn_ref[...], preferred_element_type=jnp.float32)
            + jnp.dot(h_ref[...], ws_ref[...], preferred_element_type=jnp.float32)
            + b_ref[...]
        )
        t = jnp.maximum(y, 0.0)
        if project:
            wo_ref, bo_ref = rest[0], rest[1]
            t = (
                jnp.dot(t, wo_ref[...], preferred_element_type=jnp.float32)
                + bo_ref[...]
            )
        o_ref[...] = t

    in_specs = [
        pl.BlockSpec((NC, BM, FDIM), lambda i: (0, i, 0)),
        pl.BlockSpec((BM, NW), lambda i: (i, 0)),
        pl.BlockSpec((BM, FDIM), lambda i: (i, 0)),
        pl.BlockSpec((FDIM, FDIM), lambda i: (0, 0)),
        pl.BlockSpec((FDIM, FDIM), lambda i: (0, 0)),
        pl.BlockSpec((1, FDIM), lambda i: (0, 0)),
    ]
    args = [aggp, cntp, h, wn, ws, bh.reshape(1, FDIM)]
    odim = FDIM
    if project:
        odim = wo.shape[1]
        in_specs += [
            pl.BlockSpec((FDIM, odim), lambda i: (0, 0)),
            pl.BlockSpec((1, odim), lambda i: (0, 0)),
        ]
        args += [wo, bo.reshape(1, odim)]

    return pl.pallas_call(
        body,
        grid=(N_NODES // BM,),
        in_specs=in_specs,
        out_specs=pl.BlockSpec((BM, odim), lambda i: (i, 0)),
        out_shape=jax.ShapeDtypeStruct((N_NODES, odim), jnp.float32),
    )(*args)


def kernel(x, edge_index, W_in, b_in, W_neigh, W_self, b_hidden, W_out, b_out):
    ei = edge_index.astype(jnp.int32)
    dst3d = ei[0].reshape(NW, NCHUNK, CHUNK)
    src3d = ei[1].reshape(NW, NCHUNK, CHUNK)

    h = _dense(x, W_in, b_in)
    cntp = _sc_count(dst3d)
    cnt_t = cntp.T
    aggp = _sc_aggregate(h, src3d, dst3d)
    h = _combine(aggp, cnt_t, h, W_neigh[0], W_self[0], b_hidden[0])
    aggp = _sc_aggregate(h, src3d, dst3d)
    return _combine(aggp, cnt_t, h, W_neigh[1], W_self[1], b_hidden[1],
                    W_out, b_out)
